# baseline (speedup 1.0000x reference)
import functools

import jax
import jax.numpy as jnp
from jax import lax
from jax.experimental import pallas as pl
from jax.experimental.pallas import tpu as pltpu
from jax.experimental.pallas import tpu_sc as plsc

NC = 2
NS = 16
NW = NC * NS

C = 2048


def _gather_body(idx_hbm, w_hbm, s_hbm, m_hbm, rows_out, s_out, m_out,
                 idx_v, rows_v, s_v, m_v, sem_w, sem_s, sem_m, *, rpw):
  wid = lax.axis_index("s") * NC + lax.axis_index("c")
  base0 = wid * rpw

  def chunk_body(ci, _):
    base = pl.multiple_of(base0 + ci * C, C)
    pltpu.sync_copy(idx_hbm.at[pl.ds(base, C)], idx_v)
    cw = pltpu.async_copy(w_hbm.at[idx_v], rows_v, sem_w)
    cs = pltpu.async_copy(s_hbm.at[idx_v], s_v, sem_s)
    cm = pltpu.async_copy(m_hbm.at[idx_v], m_v, sem_m)
    cw.wait()
    cs.wait()
    cm.wait()
    pltpu.sync_copy(rows_v, rows_out.at[pl.ds(base, C)])
    pltpu.sync_copy(s_v, s_out.at[pl.ds(base, C)])
    pltpu.sync_copy(m_v, m_out.at[pl.ds(base, C)])
    return ()

  lax.fori_loop(0, rpw // C, chunk_body, (), unroll=False)


def _dequant_body(q_ref, s_ref, m_ref, o_ref):
  q = q_ref[...].astype(jnp.float32)
  s = s_ref[...]
  m = m_ref[...]
  o_ref[...] = (s * (q + m)).astype(jnp.bfloat16)


def kernel(idx, weight, scales, means):
  B, L = idx.shape
  V, D = weight.shape
  BL = B * L
  rpw = BL // NW

  idxf = idx.reshape(BL)
  w32 = jax.lax.bitcast_convert_type(
      weight.reshape(V, D // 4, 4), jnp.int32)
  sf = scales.reshape(V)
  mf = means.reshape(V)

  mesh = plsc.VectorSubcoreMesh(core_axis_name="c", subcore_axis_name="s")
  rows, s_g, m_g = pl.kernel(
      functools.partial(_gather_body, rpw=rpw),
      out_type=[
          jax.ShapeDtypeStruct((BL, D // 4), jnp.int32),
          jax.ShapeDtypeStruct((BL,), jnp.float32),
          jax.ShapeDtypeStruct((BL,), jnp.float32),
      ],
      mesh=mesh,
      compiler_params=pltpu.CompilerParams(
          needs_layout_passes=False, use_tc_tiling_on_sc=False),
      scratch_types=[
          pltpu.VMEM((C,), jnp.int32),
          pltpu.VMEM((C, D // 4), jnp.int32),
          pltpu.VMEM((C,), jnp.float32),
          pltpu.VMEM((C,), jnp.float32),
          pltpu.SemaphoreType.DMA,
          pltpu.SemaphoreType.DMA,
          pltpu.SemaphoreType.DMA,
      ],
  )(idxf, w32, sf, mf)

  q8 = jax.lax.bitcast_convert_type(rows, jnp.int8).reshape(BL, D)

  Bt = 2048
  out = pl.pallas_call(
      _dequant_body,
      grid=(BL // Bt,),
      in_specs=[
          pl.BlockSpec((Bt, D), lambda i: (i, 0)),
          pl.BlockSpec((Bt, 1), lambda i: (i, 0)),
          pl.BlockSpec((Bt, 1), lambda i: (i, 0)),
      ],
      out_specs=pl.BlockSpec((Bt, D), lambda i: (i, 0)),
      out_shape=jax.ShapeDtypeStruct((BL, D), jnp.bfloat16),
  )(q8, s_g.reshape(BL, 1), m_g.reshape(BL, 1))
  return out.reshape(B, L, D)


# speedup vs baseline: 1.5913x; 1.5913x over previous
import functools

import jax
import jax.numpy as jnp
from jax import lax
from jax.experimental import pallas as pl
from jax.experimental.pallas import tpu as pltpu
from jax.experimental.pallas import tpu_sc as plsc

NC = 2
NS = 16
NW = NC * NS

C = 1024
NBUF = 2


def _gather_body(idx_hbm, w_hbm, s_hbm, m_hbm, rows_out, s_out, m_out,
                 idx_v, rows_v, s_v, m_v, sem_g, sem_w, *, rpw):
  wid = lax.axis_index("s") * NC + lax.axis_index("c")
  base0 = wid * rpw
  nchunks = rpw // C

  pending_gather = [None] * NBUF
  pending_wb = [None] * NBUF

  def start(ci, sl):
    base = pl.multiple_of(base0 + ci * C, C)
    if pending_wb[sl] is not None:
      for c in pending_wb[sl]:
        c.wait()
      pending_wb[sl] = None
    pltpu.sync_copy(idx_hbm.at[pl.ds(base, C)], idx_v.at[sl])
    pending_gather[sl] = (
        pltpu.async_copy(w_hbm.at[idx_v.at[sl]], rows_v.at[sl], sem_g.at[sl, 0]),
        pltpu.async_copy(s_hbm.at[idx_v.at[sl]], s_v.at[sl], sem_g.at[sl, 1]),
        pltpu.async_copy(m_hbm.at[idx_v.at[sl]], m_v.at[sl], sem_g.at[sl, 2]),
    )

  def drain(ci, sl):
    base = pl.multiple_of(base0 + ci * C, C)
    for c in pending_gather[sl]:
      c.wait()
    pending_gather[sl] = None
    pending_wb[sl] = (
        pltpu.async_copy(rows_v.at[sl], rows_out.at[pl.ds(base, C)], sem_w.at[sl, 0]),
        pltpu.async_copy(s_v.at[sl], s_out.at[pl.ds(base, C)], sem_w.at[sl, 1]),
        pltpu.async_copy(m_v.at[sl], m_out.at[pl.ds(base, C)], sem_w.at[sl, 2]),
    )

  start(0, 0)
  for ci in range(nchunks):
    if ci + 1 < nchunks:
      start(ci + 1, (ci + 1) % NBUF)
    drain(ci, ci % NBUF)
  for sl in range(NBUF):
    if pending_wb[sl] is not None:
      for c in pending_wb[sl]:
        c.wait()


def _dequant_body(q_ref, s_ref, m_ref, o_ref):
  q = q_ref[...].astype(jnp.float32)
  s = s_ref[...]
  m = m_ref[...]
  o_ref[...] = (s * (q + m)).astype(jnp.bfloat16)


def kernel(idx, weight, scales, means):
  B, L = idx.shape
  V, D = weight.shape
  BL = B * L
  rpw = BL // NW

  idxf = idx.reshape(BL)
  sf = scales.reshape(V)
  mf = means.reshape(V)

  mesh = plsc.VectorSubcoreMesh(core_axis_name="c", subcore_axis_name="s")
  rows, s_g, m_g = pl.kernel(
      functools.partial(_gather_body, rpw=rpw),
      out_type=[
          jax.ShapeDtypeStruct((BL, D), jnp.int8),
          jax.ShapeDtypeStruct((BL,), jnp.float32),
          jax.ShapeDtypeStruct((BL,), jnp.float32),
      ],
      mesh=mesh,
      compiler_params=pltpu.CompilerParams(
          needs_layout_passes=False, use_tc_tiling_on_sc=False),
      scratch_types=[
          pltpu.VMEM((NBUF, C), jnp.int32),
          pltpu.VMEM((NBUF, C, D), jnp.int8),
          pltpu.VMEM((NBUF, C), jnp.float32),
          pltpu.VMEM((NBUF, C), jnp.float32),
          pltpu.SemaphoreType.DMA((NBUF, 3)),
          pltpu.SemaphoreType.DMA((NBUF, 3)),
      ],
  )(idxf, weight, sf, mf)

  Bt = 2048
  out = pl.pallas_call(
      _dequant_body,
      grid=(BL // Bt,),
      in_specs=[
          pl.BlockSpec((Bt, D), lambda i: (i, 0)),
          pl.BlockSpec((Bt, 1), lambda i: (i, 0)),
          pl.BlockSpec((Bt, 1), lambda i: (i, 0)),
      ],
      out_specs=pl.BlockSpec((Bt, D), lambda i: (i, 0)),
      out_shape=jax.ShapeDtypeStruct((BL, D), jnp.bfloat16),
  )(rows, s_g.reshape(BL, 1), m_g.reshape(BL, 1))
  return out.reshape(B, L, D)
